# Initial kernel scaffold; baseline (speedup 1.0000x reference)
import jax
import jax.numpy as jnp
from jax.experimental import pallas as pl


def kernel(hidden, seq_lens, intent_index, intent_embedding, Ws_slot, as_slot, Wo_slot, ao_slot, Ws_glob, as_glob, Wo_glob, ao_glob, W1, b1, W2, b2):
    raise NotImplementedError("write your pallas kernel here")


# speedup vs baseline: 2.1785x; 2.1785x over previous
import jax
import jax.numpy as jnp
from jax.experimental import pallas as pl
from jax.experimental.pallas import tpu as pltpu

_ALPHA = 0.2
_NI = 16
_W = 2
_S = 512
_D = 256
_NH = 4
_GH = 16
_NEG = -9e15


def _leaky(x):
    return jnp.where(x > 0, x, _ALPHA * x)


def _elu(x):
    return jnp.where(x > 0, x, jnp.exp(jnp.minimum(x, 0.0)) - 1.0)


def _padded(a):
    z = jnp.zeros((_W, a.shape[1]), a.dtype)
    return jnp.concatenate([z, a, z], axis=0)


def _shift(ap, d):
    return jax.lax.slice(ap, (_W + d, 0), (_W + d + _S, ap.shape[1]))


def _band_ctx(valid_f):
    vp = _padded(valid_f)
    return [valid_f * _shift(vp, d) if d else None for d in range(-_W, _W + 1)]


def _band_logits(src, dst, masks):
    dp = _padded(dst)
    out = []
    for i, d in enumerate(range(-_W, _W + 1)):
        e = _leaky(src + _shift(dp, d))
        if d != 0:
            e = jnp.where(masks[i] > 0, e, _NEG)
        out.append(e)
    return out


def _gl_kernel(seq_ref, i0c_ref, i1c_ref, i0r_ref, i1r_ref,
               x_ref, iemb_ref,
               wsc_ref, asrc_s_ref, adst_s_ref, wos_ref, aos_src_ref, aos_dst_ref,
               wgc_ref, asrc_g_ref, adst_g_ref, wog_ref, aog_src_ref, aog_dst_ref,
               w1_ref, b1_ref, w2_ref, b2_ref,
               out_ref):
    b = pl.program_id(0)
    L = seq_ref[b]
    bf = b.astype(jnp.float32)
    x = x_ref[0]

    iota_s_col = jax.lax.broadcasted_iota(jnp.int32, (_S, 1), 0)
    valid_f = (iota_s_col < L).astype(jnp.float32)
    valid_row = (jax.lax.broadcasted_iota(jnp.int32, (1, _S), 1) < L
                 ).astype(jnp.float32)
    band_masks = _band_ctx(valid_f)

    r4 = jax.lax.broadcasted_iota(jnp.int32, (_NH, _NH * _GH), 0)
    c4 = jax.lax.broadcasted_iota(jnp.int32, (_NH, _NH * _GH), 1) // _GH
    E4 = (r4 == c4).astype(jnp.float32)

    i16r = jax.lax.broadcasted_iota(jnp.int32, (1, _NI), 1).astype(jnp.float32)
    i16c = jax.lax.broadcasted_iota(jnp.int32, (_NI, 1), 0).astype(jnp.float32)
    hit_r = ((i1c_ref[...] == i16r).astype(jnp.float32)
             * (i0c_ref[...] == bf).astype(jnp.float32))
    act_row = jnp.max(hit_r, axis=0, keepdims=True)
    hit_c = ((i16c == i1r_ref[...]).astype(jnp.float32)
             * (i0r_ref[...] == bf).astype(jnp.float32))
    act_col = jnp.max(hit_c, axis=1, keepdims=True)

    h_all = jnp.dot(x, wsc_ref[...], preferred_element_type=jnp.float32)
    src = jnp.dot(h_all, asrc_s_ref[...], preferred_element_type=jnp.float32)
    dst = jnp.dot(h_all, adst_s_ref[...], preferred_element_type=jnp.float32)
    lg = _band_logits(src, dst, band_masks)
    m = lg[0]
    for e in lg[1:]:
        m = jnp.maximum(m, e)
    ps = [jnp.exp(e - m) for e in lg]
    den = ps[0]
    for p in ps[1:]:
        den = den + p
    hp = jnp.zeros((_S, _NH * _GH), jnp.float32)
    h_p = _padded(h_all)
    for i, d in enumerate(range(-_W, _W + 1)):
        wfull = jnp.dot(ps[i] / den, E4, preferred_element_type=jnp.float32)
        hp = hp + wfull * _shift(h_p, d)
    h1 = _elu(hp)

    ho = jnp.dot(h1, wos_ref[...], preferred_element_type=jnp.float32)
    src_o = jnp.dot(ho, aos_src_ref[...], preferred_element_type=jnp.float32)
    dst_o = jnp.dot(ho, aos_dst_ref[...], preferred_element_type=jnp.float32)
    lg = _band_logits(src_o, dst_o, band_masks)
    m = lg[0]
    for e in lg[1:]:
        m = jnp.maximum(m, e)
    ps = [jnp.exp(e - m) for e in lg]
    den = ps[0]
    for p in ps[1:]:
        den = den + p
    hp_o = jnp.zeros((_S, _D), jnp.float32)
    ho_p = _padded(ho)
    for i, d in enumerate(range(-_W, _W + 1)):
        hp_o = hp_o + (ps[i] / den) * _shift(ho_p, d)
    slot_out = _elu(hp_o) + x

    hg_I = jnp.dot(iemb_ref[...], wgc_ref[...], preferred_element_type=jnp.float32)
    hg_S = jnp.dot(slot_out, wgc_ref[...], preferred_element_type=jnp.float32)
    srcg = jnp.dot(hg_S, asrc_g_ref[...], preferred_element_type=jnp.float32)
    dstg = jnp.dot(hg_S, adst_g_ref[...], preferred_element_type=jnp.float32)
    src_I = jnp.dot(hg_I, asrc_g_ref[...], preferred_element_type=jnp.float32)
    dst_I = jnp.dot(hg_I, adst_g_ref[...], preferred_element_type=jnp.float32)
    dstg_T = jnp.transpose(dstg)
    dst_I_T = jnp.transpose(dst_I)

    lg = _band_logits(srcg, dstg, band_masks)
    m = lg[0]
    for e in lg[1:]:
        m = jnp.maximum(m, e)
    int_mask = valid_f * act_row
    li = []
    mi = []
    for k in range(_NH):
        e = _leaky(srcg[:, k:k + 1] + dst_I_T[k:k + 1, :])
        e = jnp.where(int_mask > 0, e, _NEG)
        li.append(e)
        mi.append(jnp.max(e, axis=1, keepdims=True))
    m = jnp.maximum(m, jnp.concatenate(mi, axis=1))
    ps = [jnp.exp(e - m) for e in lg]
    den = ps[0]
    for p in ps[1:]:
        den = den + p
    pi = [jnp.exp(li[k] - m[:, k:k + 1]) for k in range(_NH)]
    den = den + jnp.concatenate(
        [jnp.sum(p, axis=1, keepdims=True) for p in pi], axis=1)
    hp = jnp.zeros((_S, _NH * _GH), jnp.float32)
    hg_S_p = _padded(hg_S)
    for i, d in enumerate(range(-_W, _W + 1)):
        wfull = jnp.dot(ps[i] / den, E4, preferred_element_type=jnp.float32)
        hp = hp + wfull * _shift(hg_S_p, d)
    hp_int = []
    for k in range(_NH):
        hp_int.append(jnp.dot(pi[k] / den[:, k:k + 1],
                              hg_I[:, k * _GH:(k + 1) * _GH],
                              preferred_element_type=jnp.float32))
    hp = hp + jnp.concatenate(hp_int, axis=1)
    hg1_S = _elu(hp)

    eye = (jax.lax.broadcasted_iota(jnp.int32, (_NI, _NI), 0)
           == jax.lax.broadcasted_iota(jnp.int32, (_NI, _NI), 1))
    all_II = jnp.logical_or((act_col * act_row) > 0, eye)
    all_IS = (act_col * valid_row) > 0
    hp_I = []
    for k in range(_NH):
        s_k = src_I[:, k:k + 1]
        lII = jnp.where(all_II, _leaky(s_k + dst_I_T[k:k + 1, :]), _NEG)
        lIS = jnp.where(all_IS, _leaky(s_k + dstg_T[k:k + 1, :]), _NEG)
        mI = jnp.maximum(jnp.max(lII, axis=1, keepdims=True),
                         jnp.max(lIS, axis=1, keepdims=True))
        pII = jnp.exp(lII - mI)
        pIS = jnp.exp(lIS - mI)
        denI = (jnp.sum(pII, axis=1, keepdims=True)
                + jnp.sum(pIS, axis=1, keepdims=True))
        hp_I.append(
            jnp.dot(pII / denI, hg_I[:, k * _GH:(k + 1) * _GH],
                    preferred_element_type=jnp.float32)
            + jnp.dot(pIS / denI, hg_S[:, k * _GH:(k + 1) * _GH],
                      preferred_element_type=jnp.float32))
    hg1_I = _elu(jnp.concatenate(hp_I, axis=1))

    hgo_S = jnp.dot(hg1_S, wog_ref[...], preferred_element_type=jnp.float32)
    hgo_I = jnp.dot(hg1_I, wog_ref[...], preferred_element_type=jnp.float32)
    src_go = jnp.dot(hgo_S, aog_src_ref[...], preferred_element_type=jnp.float32)
    dst_go = jnp.dot(hgo_S, aog_dst_ref[...], preferred_element_type=jnp.float32)
    dst_go_I = jnp.dot(hgo_I, aog_dst_ref[...], preferred_element_type=jnp.float32)
    dst_go_I_row = jnp.transpose(dst_go_I)
    lg = _band_logits(src_go, dst_go, band_masks)
    m = lg[0]
    for e in lg[1:]:
        m = jnp.maximum(m, e)
    ei = jnp.where(int_mask > 0, _leaky(src_go + dst_go_I_row), _NEG)
    m = jnp.maximum(m, jnp.max(ei, axis=1, keepdims=True))
    ps = [jnp.exp(e - m) for e in lg]
    den = ps[0]
    for p in ps[1:]:
        den = den + p
    pint = jnp.exp(ei - m)
    den = den + jnp.sum(pint, axis=1, keepdims=True)
    hp_go = jnp.zeros((_S, _D), jnp.float32)
    hgo_S_p = _padded(hgo_S)
    for i, d in enumerate(range(-_W, _W + 1)):
        hp_go = hp_go + (ps[i] / den) * _shift(hgo_S_p, d)
    hp_go = hp_go + jnp.dot(pint / den, hgo_I,
                            preferred_element_type=jnp.float32)
    gout_S = _elu(hp_go) + slot_out

    hf = _leaky(jnp.dot(gout_S, w1_ref[...],
                        preferred_element_type=jnp.float32) + b1_ref[...])
    out_ref[0] = jnp.dot(hf, w2_ref[...],
                         preferred_element_type=jnp.float32) + b2_ref[...]


def _full(shape):
    n = len(shape)
    return pl.BlockSpec(shape, lambda b, n=n: (0,) * n)


def kernel(hidden, seq_lens, intent_index, intent_embedding, Ws_slot, as_slot,
           Wo_slot, ao_slot, Ws_glob, as_glob, Wo_glob, ao_glob, W1, b1, W2, b2):
    B, S, D = hidden.shape
    NH, _, GH = Ws_slot.shape

    def cat_heads(Ws):
        return jnp.transpose(Ws, (1, 0, 2)).reshape(D, NH * GH)

    def blockdiag(a_heads, half):
        A = jnp.zeros((NH * GH, NH), jnp.float32)
        for k in range(NH):
            A = A.at[k * GH:(k + 1) * GH, k].set(
                a_heads[k, half * GH:(half + 1) * GH])
        return A

    ii = intent_index.astype(jnp.float32)
    i0c, i1c = ii[:, 0:1], ii[:, 1:2]
    i0r, i1r = ii[:, 0][None, :], ii[:, 1][None, :]

    args = (
        seq_lens,
        i0c, i1c, i0r, i1r,
        hidden, intent_embedding,
        cat_heads(Ws_slot), blockdiag(as_slot, 0), blockdiag(as_slot, 1),
        Wo_slot, ao_slot[:D, None], ao_slot[D:, None],
        cat_heads(Ws_glob), blockdiag(as_glob, 0), blockdiag(as_glob, 1),
        Wo_glob, ao_glob[:D, None], ao_glob[D:, None],
        W1, b1[None, :], W2, b2[None, :],
    )
    OUT = W2.shape[1]
    in_specs = [pl.BlockSpec(memory_space=pltpu.SMEM)]
    in_specs += [_full(a.shape) for a in args[1:5]]
    in_specs.append(pl.BlockSpec((1, S, D), lambda b: (b, 0, 0)))
    in_specs += [_full(a.shape) for a in args[6:]]

    return pl.pallas_call(
        _gl_kernel,
        grid=(B,),
        in_specs=in_specs,
        out_specs=pl.BlockSpec((1, S, OUT), lambda b: (b, 0, 0)),
        out_shape=jax.ShapeDtypeStruct((B, S, OUT), jnp.float32),
        compiler_params=pltpu.CompilerParams(
            dimension_semantics=("arbitrary",)),
    )(*args)


# speedup vs baseline: 3.5086x; 1.6106x over previous
import jax
import jax.numpy as jnp
from jax.experimental import pallas as pl
from jax.experimental.pallas import tpu as pltpu

_ALPHA = 0.2
_NI = 16
_W = 2
_S = 512
_D = 256
_NH = 4
_GH = 16
_NEG = -9e15


def _leaky(x):
    return jnp.where(x > 0, x, _ALPHA * x)


def _elu(x):
    return jnp.where(x > 0, x, jnp.exp(jnp.minimum(x, 0.0)) - 1.0)


def _rollr(a, d):
    return pltpu.roll(a, (-d) % _S, 1) if d else a


def _rolls(a, d):
    return pltpu.roll(a, (-d) % _S, 0) if d else a


def _dgT(a, b):
    return jax.lax.dot_general(a, b, (((0,), (0,)), ((), ())),
                               preferred_element_type=jnp.float32)


def _dgTT(a, b):
    return jax.lax.dot_general(a, b, (((0,), (1,)), ((), ())),
                               preferred_element_type=jnp.float32)


def _band_rowmasks(valid_row, iota_row):
    masks = []
    for d in range(-_W, _W + 1):
        if d == 0:
            masks.append(None)
            continue
        inr = ((iota_row + d >= 0) & (iota_row + d < _S)).astype(jnp.float32)
        m = valid_row * _rollr(valid_row, d) * inr
        masks.append((1.0 - m) * _NEG)
    return masks


def _band_softmax_rows(src_t, dst_t, masks, extra_logits=None):
    lg = []
    for i, d in enumerate(range(-_W, _W + 1)):
        e = _leaky(src_t + _rollr(dst_t, d))
        if d != 0:
            e = e + masks[i]
        lg.append(e)
    m = lg[0]
    for e in lg[1:]:
        m = jnp.maximum(m, e)
    if extra_logits is not None:
        mex = jnp.concatenate(
            [jnp.max(ex, axis=0, keepdims=True) for ex in extra_logits],
            axis=0)
        m = jnp.maximum(m, mex)
    ps = [jnp.exp(e - m) for e in lg]
    den = ps[0]
    for p in ps[1:]:
        den = den + p
    pex = None
    if extra_logits is not None:
        pex = [jnp.exp(ex - m[h:h + 1, :]) for h, ex in enumerate(extra_logits)]
        den = den + jnp.concatenate(
            [jnp.sum(p, axis=0, keepdims=True) for p in pex], axis=0)
    rden = 1.0 / den
    return [p * rden for p in ps], pex, rden


def _gl_kernel(seq_ref, i0c_ref, i1c_ref, i0r_ref, i1r_ref,
               x_ref, iemb_ref,
               wsc_ref, asrc_s_ref, adst_s_ref, wos_ref, aos_src_ref, aos_dst_ref,
               wgc_ref, asrc_g_ref, adst_g_ref, wog_ref, aog_src_ref, aog_dst_ref,
               w1_ref, b1_ref, w2_ref, b2_ref,
               out_ref):
    b = pl.program_id(0)
    L = seq_ref[b]
    bf = b.astype(jnp.float32)
    x = x_ref[0]

    iota_row = jax.lax.broadcasted_iota(jnp.int32, (1, _S), 1)
    valid_row = (iota_row < L).astype(jnp.float32)
    band_masks = _band_rowmasks(valid_row, iota_row)
    ones_row = jnp.ones((1, _D), jnp.float32)

    r4 = jax.lax.broadcasted_iota(jnp.int32, (_NH, _NH * _GH), 0)
    c4 = jax.lax.broadcasted_iota(jnp.int32, (_NH, _NH * _GH), 1) // _GH
    E4 = (r4 == c4).astype(jnp.float32)

    i16r = jax.lax.broadcasted_iota(jnp.int32, (1, _NI), 1).astype(jnp.float32)
    i16c = jax.lax.broadcasted_iota(jnp.int32, (_NI, 1), 0).astype(jnp.float32)
    hit_r = ((i1c_ref[...] == i16r).astype(jnp.float32)
             * (i0c_ref[...] == bf).astype(jnp.float32))
    act_row = jnp.max(hit_r, axis=0, keepdims=True)
    hit_c = ((i16c == i1r_ref[...]).astype(jnp.float32)
             * (i0r_ref[...] == bf).astype(jnp.float32))
    act_col = jnp.max(hit_c, axis=1, keepdims=True)

    h_all = jnp.dot(x, wsc_ref[...], preferred_element_type=jnp.float32)
    src_t = _dgTT(asrc_s_ref[...], h_all)
    dst_t = _dgTT(adst_s_ref[...], h_all)
    att, _, _ = _band_softmax_rows(src_t, dst_t, band_masks)
    hp = jnp.zeros((_S, _NH * _GH), jnp.float32)
    for i, d in enumerate(range(-_W, _W + 1)):
        hp = hp + _dgT(att[i], E4) * _rolls(h_all, d)
    h1 = _elu(hp)

    ho = jnp.dot(h1, wos_ref[...], preferred_element_type=jnp.float32)
    src_t = _dgTT(aos_src_ref[...], ho)
    dst_t = _dgTT(aos_dst_ref[...], ho)
    att, _, _ = _band_softmax_rows(src_t, dst_t, band_masks)
    hp_o = jnp.zeros((_S, _D), jnp.float32)
    for i, d in enumerate(range(-_W, _W + 1)):
        hp_o = hp_o + _dgT(att[i], ones_row) * _rolls(ho, d)
    slot_out = _elu(hp_o) + x

    hg_I = jnp.dot(iemb_ref[...], wgc_ref[...], preferred_element_type=jnp.float32)
    hg_S = jnp.dot(slot_out, wgc_ref[...], preferred_element_type=jnp.float32)
    srcg_t = _dgTT(asrc_g_ref[...], hg_S)
    dstg_t = _dgTT(adst_g_ref[...], hg_S)
    src_I = jnp.dot(hg_I, asrc_g_ref[...], preferred_element_type=jnp.float32)
    dst_I = jnp.dot(hg_I, adst_g_ref[...], preferred_element_type=jnp.float32)
    dst_I_t = _dgTT(adst_g_ref[...], hg_I)

    nm_IS = (1.0 - act_col * valid_row) * _NEG
    ints = []
    for k in range(_NH):
        ints.append(_leaky(srcg_t[k:k + 1, :] + dst_I[:, k:k + 1]) + nm_IS)
    att, pint, rden = _band_softmax_rows(srcg_t, dstg_t, band_masks, ints)
    hp = jnp.zeros((_S, _NH * _GH), jnp.float32)
    for i, d in enumerate(range(-_W, _W + 1)):
        hp = hp + _dgT(att[i], E4) * _rolls(hg_S, d)
    hp_int = []
    for k in range(_NH):
        hp_int.append(_dgT(pint[k] * rden[k:k + 1, :],
                           hg_I[:, k * _GH:(k + 1) * _GH]))
    hp = hp + jnp.concatenate(hp_int, axis=1)
    hg1_S = _elu(hp)

    eye = (jax.lax.broadcasted_iota(jnp.int32, (_NI, _NI), 0)
           == jax.lax.broadcasted_iota(jnp.int32, (_NI, _NI), 1))
    nm_II = jnp.where(
        jnp.logical_or((act_col * act_row) > 0, eye), 0.0, _NEG)
    hp_I = []
    for k in range(_NH):
        s_k = src_I[:, k:k + 1]
        lII = _leaky(s_k + dst_I_t[k:k + 1, :]) + nm_II
        lIS = _leaky(s_k + dstg_t[k:k + 1, :]) + nm_IS
        mI = jnp.maximum(jnp.max(lII, axis=1, keepdims=True),
                         jnp.max(lIS, axis=1, keepdims=True))
        pII = jnp.exp(lII - mI)
        pIS = jnp.exp(lIS - mI)
        rdenI = 1.0 / (jnp.sum(pII, axis=1, keepdims=True)
                       + jnp.sum(pIS, axis=1, keepdims=True))
        hp_I.append(
            jnp.dot(pII * rdenI, hg_I[:, k * _GH:(k + 1) * _GH],
                    preferred_element_type=jnp.float32)
            + jnp.dot(pIS * rdenI, hg_S[:, k * _GH:(k + 1) * _GH],
                      preferred_element_type=jnp.float32))
    hg1_I = _elu(jnp.concatenate(hp_I, axis=1))

    hgo_S = jnp.dot(hg1_S, wog_ref[...], preferred_element_type=jnp.float32)
    hgo_I = jnp.dot(hg1_I, wog_ref[...], preferred_element_type=jnp.float32)
    src_t = _dgTT(aog_src_ref[...], hgo_S)
    dst_t = _dgTT(aog_dst_ref[...], hgo_S)
    dst_go_I = jnp.dot(hgo_I, aog_dst_ref[...], preferred_element_type=jnp.float32)
    ei = _leaky(src_t + dst_go_I) + nm_IS
    att, pint, rden = _band_softmax_rows(src_t, dst_t, band_masks, [ei])
    hp_go = jnp.zeros((_S, _D), jnp.float32)
    for i, d in enumerate(range(-_W, _W + 1)):
        hp_go = hp_go + _dgT(att[i], ones_row) * _rolls(hgo_S, d)
    hp_go = hp_go + _dgT(pint[0] * rden, hgo_I)
    gout_S = _elu(hp_go) + slot_out

    hf = _leaky(jnp.dot(gout_S, w1_ref[...],
                        preferred_element_type=jnp.float32) + b1_ref[...])
    out_ref[0] = jnp.dot(hf, w2_ref[...],
                         preferred_element_type=jnp.float32) + b2_ref[...]


def _full(shape):
    n = len(shape)
    return pl.BlockSpec(shape, lambda b, n=n: (0,) * n)


def kernel(hidden, seq_lens, intent_index, intent_embedding, Ws_slot, as_slot,
           Wo_slot, ao_slot, Ws_glob, as_glob, Wo_glob, ao_glob, W1, b1, W2, b2):
    B, S, D = hidden.shape
    NH, _, GH = Ws_slot.shape

    def cat_heads(Ws):
        return jnp.transpose(Ws, (1, 0, 2)).reshape(D, NH * GH)

    def blockdiag(a_heads, half):
        A = jnp.zeros((NH * GH, NH), jnp.float32)
        for k in range(NH):
            A = A.at[k * GH:(k + 1) * GH, k].set(
                a_heads[k, half * GH:(half + 1) * GH])
        return A

    ii = intent_index.astype(jnp.float32)
    i0c, i1c = ii[:, 0:1], ii[:, 1:2]
    i0r, i1r = ii[:, 0][None, :], ii[:, 1][None, :]

    args = (
        seq_lens,
        i0c, i1c, i0r, i1r,
        hidden, intent_embedding,
        cat_heads(Ws_slot), blockdiag(as_slot, 0), blockdiag(as_slot, 1),
        Wo_slot, ao_slot[:D, None], ao_slot[D:, None],
        cat_heads(Ws_glob), blockdiag(as_glob, 0), blockdiag(as_glob, 1),
        Wo_glob, ao_glob[:D, None], ao_glob[D:, None],
        W1, b1[None, :], W2, b2[None, :],
    )
    OUT = W2.shape[1]
    in_specs = [pl.BlockSpec(memory_space=pltpu.SMEM)]
    in_specs += [_full(a.shape) for a in args[1:5]]
    in_specs.append(pl.BlockSpec((1, S, D), lambda b: (b, 0, 0)))
    in_specs += [_full(a.shape) for a in args[6:]]

    return pl.pallas_call(
        _gl_kernel,
        grid=(B,),
        in_specs=in_specs,
        out_specs=pl.BlockSpec((1, S, OUT), lambda b: (b, 0, 0)),
        out_shape=jax.ShapeDtypeStruct((B, S, OUT), jnp.float32),
        compiler_params=pltpu.CompilerParams(
            dimension_semantics=("arbitrary",)),
    )(*args)


# speedup vs baseline: 3.7804x; 1.0775x over previous
import jax
import jax.numpy as jnp
from jax.experimental import pallas as pl
from jax.experimental.pallas import tpu as pltpu

_ALPHA = 0.2
_NI = 16
_W = 2
_S = 512
_D = 256
_NH = 4
_GH = 16
_NEG = -9e15


def _leaky(x):
    return jnp.where(x > 0, x, _ALPHA * x)


def _elu(x):
    return jnp.where(x > 0, x, jnp.exp(jnp.minimum(x, 0.0)) - 1.0)


def _rollr(a, d):
    return pltpu.roll(a, (-d) % _S, 1) if d else a


def _dot(a, b):
    return jnp.dot(a, b, preferred_element_type=jnp.float32)


def _dgT(a, b):
    return jax.lax.dot_general(a, b, (((0,), (0,)), ((), ())),
                               preferred_element_type=jnp.float32)


def _dgNT(a, b):
    return jax.lax.dot_general(a, b, (((1,), (1,)), ((), ())),
                               preferred_element_type=jnp.float32)


def _band_rowmasks(valid_row, iota_row):
    masks = []
    for d in range(-_W, _W + 1):
        if d == 0:
            masks.append(None)
            continue
        inr = ((iota_row + d >= 0) & (iota_row + d < _S)).astype(jnp.float32)
        m = valid_row * _rollr(valid_row, d) * inr
        masks.append((1.0 - m) * _NEG)
    return masks


def _band_softmax_rows(src_t, dst_t, masks, extra_logits=None):
    lg = []
    for i, d in enumerate(range(-_W, _W + 1)):
        e = _leaky(src_t + _rollr(dst_t, d))
        if d != 0:
            e = e + masks[i]
        lg.append(e)
    m = lg[0]
    for e in lg[1:]:
        m = jnp.maximum(m, e)
    if extra_logits is not None:
        mex = jnp.concatenate(
            [jnp.max(ex, axis=0, keepdims=True) for ex in extra_logits],
            axis=0)
        m = jnp.maximum(m, mex)
    ps = [jnp.exp(e - m) for e in lg]
    den = ps[0]
    for p in ps[1:]:
        den = den + p
    pex = None
    if extra_logits is not None:
        pex = [jnp.exp(ex - m[h:h + 1, :]) for h, ex in enumerate(extra_logits)]
        den = den + jnp.concatenate(
            [jnp.sum(p, axis=0, keepdims=True) for p in pex], axis=0)
    rden = 1.0 / den
    return [p * rden for p in ps], pex, rden


def _gl_kernel(seq_ref, i0c_ref, i1c_ref, i0r_ref, i1r_ref,
               x_ref, iembt_ref,
               wsct_ref, asrcst_ref, adstst_ref, wost_ref, aossr_ref, aosdr_ref,
               wgct_ref, asrcgt_ref, adstgt_ref, asrcg_ref, adstg_ref,
               wogt_ref, aogsr_ref, aogdr_ref, aogdc_ref,
               w1t_ref, b1c_ref, w2_ref, b2r_ref,
               out_ref):
    b = pl.program_id(0)
    L = seq_ref[b]
    bf = b.astype(jnp.float32)
    x_t = jnp.transpose(x_ref[0])

    iota_row = jax.lax.broadcasted_iota(jnp.int32, (1, _S), 1)
    valid_row = (iota_row < L).astype(jnp.float32)
    band_masks = _band_rowmasks(valid_row, iota_row)

    r4 = jax.lax.broadcasted_iota(jnp.int32, (_NH * _GH, _NH), 0) // _GH
    c4 = jax.lax.broadcasted_iota(jnp.int32, (_NH * _GH, _NH), 1)
    E4T = (r4 == c4).astype(jnp.float32)

    i16r = jax.lax.broadcasted_iota(jnp.int32, (1, _NI), 1).astype(jnp.float32)
    i16c = jax.lax.broadcasted_iota(jnp.int32, (_NI, 1), 0).astype(jnp.float32)
    hit_r = ((i1c_ref[...] == i16r).astype(jnp.float32)
             * (i0c_ref[...] == bf).astype(jnp.float32))
    act_row = jnp.max(hit_r, axis=0, keepdims=True)
    hit_c = ((i16c == i1r_ref[...]).astype(jnp.float32)
             * (i0r_ref[...] == bf).astype(jnp.float32))
    act_col = jnp.max(hit_c, axis=1, keepdims=True)

    h_all_t = _dot(wsct_ref[...], x_t)
    src_t = _dot(asrcst_ref[...], h_all_t)
    dst_t = _dot(adstst_ref[...], h_all_t)
    att, _, _ = _band_softmax_rows(src_t, dst_t, band_masks)
    hp_t = jnp.zeros((_NH * _GH, _S), jnp.float32)
    for i, d in enumerate(range(-_W, _W + 1)):
        hp_t = hp_t + _dot(E4T, att[i]) * _rollr(h_all_t, d)
    h1_t = _elu(hp_t)

    ho_t = _dot(wost_ref[...], h1_t)
    src_t = _dot(aossr_ref[...], ho_t)
    dst_t = _dot(aosdr_ref[...], ho_t)
    att, _, _ = _band_softmax_rows(src_t, dst_t, band_masks)
    hp_o = jnp.zeros((_D, _S), jnp.float32)
    for i, d in enumerate(range(-_W, _W + 1)):
        hp_o = hp_o + att[i] * _rollr(ho_t, d)
    slot_out_t = _elu(hp_o) + x_t

    hg_I_t = _dot(wgct_ref[...], iembt_ref[...])
    hg_S_t = _dot(wgct_ref[...], slot_out_t)
    srcg_t = _dot(asrcgt_ref[...], hg_S_t)
    dstg_t = _dot(adstgt_ref[...], hg_S_t)
    src_I = _dgT(hg_I_t, asrcg_ref[...])
    dst_I = _dgT(hg_I_t, adstg_ref[...])
    dst_I_t = _dot(adstgt_ref[...], hg_I_t)

    nm_IS = (1.0 - act_col * valid_row) * _NEG
    ints = []
    for k in range(_NH):
        ints.append(_leaky(srcg_t[k:k + 1, :] + dst_I[:, k:k + 1]) + nm_IS)
    att, pint, rden = _band_softmax_rows(srcg_t, dstg_t, band_masks, ints)
    hp_t = jnp.zeros((_NH * _GH, _S), jnp.float32)
    for i, d in enumerate(range(-_W, _W + 1)):
        hp_t = hp_t + _dot(E4T, att[i]) * _rollr(hg_S_t, d)
    hp_int = []
    for k in range(_NH):
        hp_int.append(_dot(hg_I_t[k * _GH:(k + 1) * _GH, :],
                           pint[k] * rden[k:k + 1, :]))
    hp_t = hp_t + jnp.concatenate(hp_int, axis=0)
    hg1_S_t = _elu(hp_t)

    eye = (jax.lax.broadcasted_iota(jnp.int32, (_NI, _NI), 0)
           == jax.lax.broadcasted_iota(jnp.int32, (_NI, _NI), 1))
    nm_II = jnp.where(
        jnp.logical_or((act_col * act_row) > 0, eye), 0.0, _NEG)
    hp_I = []
    for k in range(_NH):
        s_k = src_I[:, k:k + 1]
        lII = _leaky(s_k + dst_I_t[k:k + 1, :]) + nm_II
        lIS = _leaky(s_k + dstg_t[k:k + 1, :]) + nm_IS
        mI = jnp.maximum(jnp.max(lII, axis=1, keepdims=True),
                         jnp.max(lIS, axis=1, keepdims=True))
        pII = jnp.exp(lII - mI)
        pIS = jnp.exp(lIS - mI)
        rdenI = 1.0 / (jnp.sum(pII, axis=1, keepdims=True)
                       + jnp.sum(pIS, axis=1, keepdims=True))
        hp_I.append(_dgNT(hg_I_t[k * _GH:(k + 1) * _GH, :], pII * rdenI)
                    + _dgNT(hg_S_t[k * _GH:(k + 1) * _GH, :], pIS * rdenI))
    hg1_I_t = _elu(jnp.concatenate(hp_I, axis=0))

    hgo_S_t = _dot(wogt_ref[...], hg1_S_t)
    hgo_I_t = _dot(wogt_ref[...], hg1_I_t)
    src_t = _dot(aogsr_ref[...], hgo_S_t)
    dst_t = _dot(aogdr_ref[...], hgo_S_t)
    dst_go_I = _dgT(hgo_I_t, aogdc_ref[...])
    ei = _leaky(src_t + dst_go_I) + nm_IS
    att, pint, rden = _band_softmax_rows(src_t, dst_t, band_masks, [ei])
    hp_go = jnp.zeros((_D, _S), jnp.float32)
    for i, d in enumerate(range(-_W, _W + 1)):
        hp_go = hp_go + att[i] * _rollr(hgo_S_t, d)
    hp_go = hp_go + _dot(hgo_I_t, pint[0] * rden)
    gout_t = _elu(hp_go) + slot_out_t

    hf_t = _leaky(_dot(w1t_ref[...], gout_t) + b1c_ref[...])
    out_ref[0] = _dgT(hf_t, w2_ref[...]) + b2r_ref[...]


def _full(shape):
    n = len(shape)
    return pl.BlockSpec(shape, lambda b, n=n: (0,) * n)


def kernel(hidden, seq_lens, intent_index, intent_embedding, Ws_slot, as_slot,
           Wo_slot, ao_slot, Ws_glob, as_glob, Wo_glob, ao_glob, W1, b1, W2, b2):
    B, S, D = hidden.shape
    NH, _, GH = Ws_slot.shape

    def cat_heads_t(Ws):
        return jnp.transpose(Ws, (1, 0, 2)).reshape(D, NH * GH).T

    def blockdiag(a_heads, half):
        A = jnp.zeros((NH * GH, NH), jnp.float32)
        for k in range(NH):
            A = A.at[k * GH:(k + 1) * GH, k].set(
                a_heads[k, half * GH:(half + 1) * GH])
        return A

    ii = intent_index.astype(jnp.float32)
    i0c, i1c = ii[:, 0:1], ii[:, 1:2]
    i0r, i1r = ii[:, 0][None, :], ii[:, 1][None, :]

    args = (
        seq_lens,
        i0c, i1c, i0r, i1r,
        hidden, intent_embedding.T,
        cat_heads_t(Ws_slot), blockdiag(as_slot, 0).T, blockdiag(as_slot, 1).T,
        Wo_slot.T, ao_slot[None, :D], ao_slot[None, D:],
        cat_heads_t(Ws_glob), blockdiag(as_glob, 0).T, blockdiag(as_glob, 1).T,
        blockdiag(as_glob, 0), blockdiag(as_glob, 1),
        Wo_glob.T, ao_glob[None, :D], ao_glob[None, D:], ao_glob[D:, None],
        W1.T, b1[:, None], W2, b2[None, :],
    )
    OUT = W2.shape[1]
    in_specs = [pl.BlockSpec(memory_space=pltpu.SMEM)]
    in_specs += [_full(a.shape) for a in args[1:5]]
    in_specs.append(pl.BlockSpec((1, S, D), lambda b: (b, 0, 0)))
    in_specs += [_full(a.shape) for a in args[6:]]

    return pl.pallas_call(
        _gl_kernel,
        grid=(B,),
        in_specs=in_specs,
        out_specs=pl.BlockSpec((1, S, OUT), lambda b: (b, 0, 0)),
        out_shape=jax.ShapeDtypeStruct((B, S, OUT), jnp.float32),
        compiler_params=pltpu.CompilerParams(
            dimension_semantics=("arbitrary",)),
    )(*args)


# speedup vs baseline: 4.0845x; 1.0804x over previous
import jax
import jax.numpy as jnp
from jax.experimental import pallas as pl
from jax.experimental.pallas import tpu as pltpu

_ALPHA = 0.2
_NI = 16
_W = 2
_S = 512
_D = 256
_NH = 4
_GH = 16
_NEG = -9e15


def _leaky(x):
    return jnp.where(x > 0, x, _ALPHA * x)


def _elu(x):
    return jnp.where(x > 0, x, jnp.exp(jnp.minimum(x, 0.0)) - 1.0)


def _rollr(a, d):
    return pltpu.roll(a, (-d) % _S, 1) if d else a


def _dot(a, b):
    return jnp.dot(a, b, preferred_element_type=jnp.float32)


def _dgT(a, b):
    return jax.lax.dot_general(a, b, (((0,), (0,)), ((), ())),
                               preferred_element_type=jnp.float32)


def _dgTT(a, b):
    return jax.lax.dot_general(a, b, (((0,), (1,)), ((), ())),
                               preferred_element_type=jnp.float32)


def _dgNT(a, b):
    return jax.lax.dot_general(a, b, (((1,), (1,)), ((), ())),
                               preferred_element_type=jnp.float32)


def _band_rowmasks(valid_row, iota_row):
    masks = []
    for d in range(-_W, _W + 1):
        if d == 0:
            masks.append(None)
            continue
        inr = ((iota_row + d >= 0) & (iota_row + d < _S)).astype(jnp.float32)
        m = valid_row * _rollr(valid_row, d) * inr
        masks.append((1.0 - m) * _NEG)
    return masks


def _band_softmax_rows(src_t, dst_t, masks, extra_logits=None):
    lg = []
    for i, d in enumerate(range(-_W, _W + 1)):
        e = _leaky(src_t + _rollr(dst_t, d))
        if d != 0:
            e = e + masks[i]
        lg.append(e)
    m = lg[0]
    for e in lg[1:]:
        m = jnp.maximum(m, e)
    if extra_logits is not None:
        mex = jnp.concatenate(
            [jnp.max(ex, axis=0, keepdims=True) for ex in extra_logits],
            axis=0)
        m = jnp.maximum(m, mex)
    ps = [jnp.exp(e - m) for e in lg]
    den = ps[0]
    for p in ps[1:]:
        den = den + p
    pex = None
    if extra_logits is not None:
        pex = [jnp.exp(ex - m[h:h + 1, :]) for h, ex in enumerate(extra_logits)]
        den = den + jnp.concatenate(
            [jnp.sum(p, axis=0, keepdims=True) for p in pex], axis=0)
    rden = 1.0 / den
    return [p * rden for p in ps], pex, rden


def _gl_kernel(seq_ref, idx_ref,
               x_ref, iemb_ref,
               wss_ref, ass_ref, wos_ref, aos_ref,
               wsg_ref, asg_ref, wog_ref, aog_ref,
               w1_ref, b1_ref, w2_ref, b2_ref,
               out_ref):
    b = pl.program_id(0)
    L = seq_ref[b]
    x_t = jnp.transpose(x_ref[0])

    iota_row = jax.lax.broadcasted_iota(jnp.int32, (1, _S), 1)
    valid_row = (iota_row < L).astype(jnp.float32)
    band_masks = _band_rowmasks(valid_row, iota_row)

    r4 = jax.lax.broadcasted_iota(jnp.int32, (_NH * _GH, _NH), 0) // _GH
    c4 = jax.lax.broadcasted_iota(jnp.int32, (_NH * _GH, _NH), 1)
    E4T = (r4 == c4).astype(jnp.float32)

    i16r = jax.lax.broadcasted_iota(jnp.int32, (1, _NI), 1)
    hit = jnp.logical_and(idx_ref[:, 1:2] == i16r,
                          idx_ref[:, 0:1] == b)
    act_row = jnp.max(hit.astype(jnp.float32), axis=0, keepdims=True)
    act_col = jnp.transpose(act_row)

    h_t = [_dgT(wss_ref[k], x_t) for k in range(_NH)]
    h_all_t = jnp.concatenate(h_t, axis=0)
    src_t = jnp.concatenate(
        [_dot(ass_ref[k:k + 1, :_GH], h_t[k]) for k in range(_NH)], axis=0)
    dst_t = jnp.concatenate(
        [_dot(ass_ref[k:k + 1, _GH:], h_t[k]) for k in range(_NH)], axis=0)
    att, _, _ = _band_softmax_rows(src_t, dst_t, band_masks)
    hp_t = jnp.zeros((_NH * _GH, _S), jnp.float32)
    for i, d in enumerate(range(-_W, _W + 1)):
        hp_t = hp_t + _dot(E4T, att[i]) * _rollr(h_all_t, d)
    h1_t = _elu(hp_t)

    ho_t = _dgT(wos_ref[...], h1_t)
    src_t = _dot(aos_ref[:, :_D], ho_t)
    dst_t = _dot(aos_ref[:, _D:], ho_t)
    att, _, _ = _band_softmax_rows(src_t, dst_t, band_masks)
    hp_o = jnp.zeros((_D, _S), jnp.float32)
    for i, d in enumerate(range(-_W, _W + 1)):
        hp_o = hp_o + att[i] * _rollr(ho_t, d)
    slot_out_t = _elu(hp_o) + x_t

    hgI_t = [_dgTT(wsg_ref[k], iemb_ref[...]) for k in range(_NH)]
    hg_I_t = jnp.concatenate(hgI_t, axis=0)
    hgS_t = [_dgT(wsg_ref[k], slot_out_t) for k in range(_NH)]
    hg_S_t = jnp.concatenate(hgS_t, axis=0)
    srcg_t = jnp.concatenate(
        [_dot(asg_ref[k:k + 1, :_GH], hgS_t[k]) for k in range(_NH)], axis=0)
    dstg_t = jnp.concatenate(
        [_dot(asg_ref[k:k + 1, _GH:], hgS_t[k]) for k in range(_NH)], axis=0)
    srcI_c = [_dgTT(hgI_t[k], asg_ref[k:k + 1, :_GH]) for k in range(_NH)]
    dstI_c = [_dgTT(hgI_t[k], asg_ref[k:k + 1, _GH:]) for k in range(_NH)]
    dstI_r = [_dot(asg_ref[k:k + 1, _GH:], hgI_t[k]) for k in range(_NH)]

    nm_IS = (1.0 - act_col * valid_row) * _NEG
    ints = []
    for k in range(_NH):
        ints.append(_leaky(srcg_t[k:k + 1, :] + dstI_c[k]) + nm_IS)
    att, pint, rden = _band_softmax_rows(srcg_t, dstg_t, band_masks, ints)
    hp_t = jnp.zeros((_NH * _GH, _S), jnp.float32)
    for i, d in enumerate(range(-_W, _W + 1)):
        hp_t = hp_t + _dot(E4T, att[i]) * _rollr(hg_S_t, d)
    hp_int = [_dot(hgI_t[k], pint[k] * rden[k:k + 1, :]) for k in range(_NH)]
    hp_t = hp_t + jnp.concatenate(hp_int, axis=0)
    hg1_S_t = _elu(hp_t)

    eye = (jax.lax.broadcasted_iota(jnp.int32, (_NI, _NI), 0)
           == jax.lax.broadcasted_iota(jnp.int32, (_NI, _NI), 1))
    nm_II = jnp.where(
        jnp.logical_or((act_col * act_row) > 0, eye), 0.0, _NEG)
    hp_I = []
    for k in range(_NH):
        lII = _leaky(srcI_c[k] + dstI_r[k]) + nm_II
        lIS = _leaky(srcI_c[k] + dstg_t[k:k + 1, :]) + nm_IS
        mI = jnp.maximum(jnp.max(lII, axis=1, keepdims=True),
                         jnp.max(lIS, axis=1, keepdims=True))
        pII = jnp.exp(lII - mI)
        pIS = jnp.exp(lIS - mI)
        rdenI = 1.0 / (jnp.sum(pII, axis=1, keepdims=True)
                       + jnp.sum(pIS, axis=1, keepdims=True))
        hp_I.append(_dgNT(hgI_t[k], pII * rdenI)
                    + _dgNT(hgS_t[k], pIS * rdenI))
    hg1_I_t = _elu(jnp.concatenate(hp_I, axis=0))

    hgo_S_t = _dgT(wog_ref[...], hg1_S_t)
    hgo_I_t = _dgT(wog_ref[...], hg1_I_t)
    src_t = _dot(aog_ref[:, :_D], hgo_S_t)
    dst_t = _dot(aog_ref[:, _D:], hgo_S_t)
    dst_go_I = _dgTT(hgo_I_t, aog_ref[:, _D:])
    ei = _leaky(src_t + dst_go_I) + nm_IS
    att, pint, rden = _band_softmax_rows(src_t, dst_t, band_masks, [ei])
    hp_go = jnp.zeros((_D, _S), jnp.float32)
    for i, d in enumerate(range(-_W, _W + 1)):
        hp_go = hp_go + att[i] * _rollr(hgo_S_t, d)
    hp_go = hp_go + _dot(hgo_I_t, pint[0] * rden)
    gout_t = _elu(hp_go) + slot_out_t

    b1_c = jnp.transpose(b1_ref[...])
    hf_t = _leaky(_dgT(w1_ref[...], gout_t) + b1_c)
    out_ref[0] = _dgT(hf_t, w2_ref[...]) + b2_ref[...]


def _full(shape):
    n = len(shape)
    return pl.BlockSpec(shape, lambda b, n=n: (0,) * n)


def kernel(hidden, seq_lens, intent_index, intent_embedding, Ws_slot, as_slot,
           Wo_slot, ao_slot, Ws_glob, as_glob, Wo_glob, ao_glob, W1, b1, W2, b2):
    B, S, D = hidden.shape
    OUT = W2.shape[1]

    args = (
        seq_lens, intent_index,
        hidden, intent_embedding,
        Ws_slot, as_slot, Wo_slot, ao_slot[None, :],
        Ws_glob, as_glob, Wo_glob, ao_glob[None, :],
        W1, b1[None, :], W2, b2[None, :],
    )
    in_specs = [pl.BlockSpec(memory_space=pltpu.SMEM)]
    in_specs.append(_full(intent_index.shape))
    in_specs.append(pl.BlockSpec((1, S, D), lambda b: (b, 0, 0)))
    in_specs += [_full(a.shape) for a in args[3:]]

    return pl.pallas_call(
        _gl_kernel,
        grid=(B,),
        in_specs=in_specs,
        out_specs=pl.BlockSpec((1, S, OUT), lambda b: (b, 0, 0)),
        out_shape=jax.ShapeDtypeStruct((B, S, OUT), jnp.float32),
        compiler_params=pltpu.CompilerParams(
            dimension_semantics=("arbitrary",)),
    )(*args)
